# Initial kernel scaffold; baseline (speedup 1.0000x reference)
import jax
import jax.numpy as jnp
from jax.experimental import pallas as pl


def kernel(x, kernel_basis, fiber_kernel_basis, edge_index, Wk, conv_bias, ln_g, ln_b, W1, b1, W2, b2, layer_scale):
    raise NotImplementedError("write your pallas kernel here")


# speedup vs baseline: 2.2337x; 2.2337x over previous
import functools

import jax
import jax.numpy as jnp
from jax import lax
from jax.experimental import pallas as pl
from jax.experimental.pallas import tpu as pltpu
from jax.experimental.pallas import tpu_sc as plsc

N = 10000
E = 320000
C = 128
K = 16
WF = 4 * C

NC = 2
NS = 16
NW = NC * NS
EPW = E // NW
CH = 80
NCHUNK = EPW // CH
N2 = 10240
RPT = N2 // NS
ZB = 128

_LANES = C // 16


def _edge_kernel(kernel_basis, Wk):
    e8 = E // 8
    basis2 = kernel_basis.reshape(e8, 8 * K)
    wbig = jnp.kron(jnp.eye(8, dtype=jnp.float32), Wk.T)

    def body(a_ref, w_ref, o_ref):
        o_ref[...] = jnp.dot(a_ref[...], w_ref[...],
                             preferred_element_type=jnp.float32)

    mb = 2000
    out = pl.pallas_call(
        body,
        grid=(e8 // mb,),
        in_specs=[
            pl.BlockSpec((mb, 8 * K), lambda i: (i, 0)),
            pl.BlockSpec((8 * K, 8 * C), lambda i: (0, 0)),
        ],
        out_specs=pl.BlockSpec((mb, 8 * C), lambda i: (i, 0)),
        out_shape=jax.ShapeDtypeStruct((e8, 8 * C), jnp.float32),
    )(basis2, wbig)
    return out.reshape(E, C)


def _sc_segment(x, kern_full, src, dst):
    mesh = plsc.VectorSubcoreMesh(core_axis_name="c", subcore_axis_name="s")

    @functools.partial(
        pl.kernel,
        out_type=jax.ShapeDtypeStruct((NC, N2, C), jnp.float32),
        mesh=mesh,
        scratch_types=[
            pltpu.VMEM((CH,), jnp.int32),
            pltpu.VMEM((CH,), jnp.int32),
            pltpu.VMEM((CH, C), jnp.float32),
            pltpu.VMEM((CH, C), jnp.float32),
            pltpu.VMEM((ZB, C), jnp.float32),
            pltpu.VMEM_SHARED((N2, C), jnp.float32),
            pltpu.SemaphoreType.DMA,
        ],
    )
    def sc(x_hbm, kern_hbm, src_hbm, dst_hbm, out_hbm,
           src_v, dst_v, xs_v, kern_v, zero_v, acc_sh, sem):
        cid = lax.axis_index("c")
        sid = lax.axis_index("s")
        wid = sid * NC + cid

        z16 = jnp.zeros((16,), jnp.float32)

        def zrow(i, carry):
            for c in range(_LANES):
                zero_v[i, pl.ds(c * 16, 16)] = z16
            return carry

        lax.fori_loop(0, ZB, zrow, 0)
        for r in range(RPT // ZB):
            pltpu.sync_copy(zero_v, acc_sh.at[pl.ds(sid * RPT + r * ZB, ZB)])
        plsc.subcore_barrier()

        def chunk(j, carry):
            base = wid * EPW + j * CH
            pltpu.sync_copy(src_hbm.at[pl.ds(base, CH)], src_v)
            pltpu.sync_copy(dst_hbm.at[pl.ds(base, CH)], dst_v)
            pltpu.sync_copy(kern_hbm.at[pl.ds(base, CH)], kern_v)
            pltpu.async_copy(x_hbm.at[src_v], xs_v, sem).wait()

            def erow(e, c2):
                for c in range(_LANES):
                    sl = pl.ds(c * 16, 16)
                    xs_v[e, sl] = xs_v[e, sl] * kern_v[e, sl]
                return c2

            lax.fori_loop(0, CH, erow, 0)
            pltpu.sync_copy(xs_v, acc_sh.at[dst_v], add=True)
            return carry

        lax.fori_loop(0, NCHUNK, chunk, 0)
        plsc.subcore_barrier()

        for r in range(RPT // ZB):
            off = sid * RPT + r * ZB
            pltpu.sync_copy(acc_sh.at[pl.ds(off, ZB)],
                            out_hbm.at[cid, pl.ds(off, ZB)])

    return sc(x, kern_full, src, dst)


def _ln_mlp(partials, x, conv_bias, ln_g, ln_b, W1T, b1, W2T, b2, layer_scale):
    nb = 1000

    def body(p_ref, x_ref, cb_ref, g_ref, b_ref, w1_ref, b1_ref,
             w2_ref, b2_ref, ls_ref, o_ref):
        x1 = p_ref[0] + p_ref[1] + cb_ref[...]
        mu = jnp.mean(x1, axis=-1, keepdims=True)
        xc = x1 - mu
        var = jnp.mean(xc * xc, axis=-1, keepdims=True)
        h = xc * lax.rsqrt(var + 1e-5) * g_ref[...] + b_ref[...]
        a = jnp.dot(h, w1_ref[...], preferred_element_type=jnp.float32)
        a = a + b1_ref[...]
        a = 0.5 * a * (1.0 + lax.erf(a * 0.7071067811865476))
        o = jnp.dot(a, w2_ref[...], preferred_element_type=jnp.float32)
        o = o + b2_ref[...]
        o_ref[...] = ls_ref[...] * o + x_ref[...]

    return pl.pallas_call(
        body,
        grid=(N // nb,),
        in_specs=[
            pl.BlockSpec((NC, nb, C), lambda i: (0, i, 0)),
            pl.BlockSpec((nb, C), lambda i: (i, 0)),
            pl.BlockSpec((1, C), lambda i: (0, 0)),
            pl.BlockSpec((1, C), lambda i: (0, 0)),
            pl.BlockSpec((1, C), lambda i: (0, 0)),
            pl.BlockSpec((C, WF), lambda i: (0, 0)),
            pl.BlockSpec((1, WF), lambda i: (0, 0)),
            pl.BlockSpec((WF, C), lambda i: (0, 0)),
            pl.BlockSpec((1, C), lambda i: (0, 0)),
            pl.BlockSpec((1, C), lambda i: (0, 0)),
        ],
        out_specs=pl.BlockSpec((nb, C), lambda i: (i, 0)),
        out_shape=jax.ShapeDtypeStruct((N, C), jnp.float32),
    )(partials, x, conv_bias.reshape(1, C), ln_g.reshape(1, C),
      ln_b.reshape(1, C), W1T, b1.reshape(1, WF), W2T, b2.reshape(1, C),
      layer_scale.reshape(1, C))


def kernel(x, kernel_basis, fiber_kernel_basis, edge_index, Wk, conv_bias,
           ln_g, ln_b, W1, b1, W2, b2, layer_scale):
    src = edge_index[0]
    dst = edge_index[1]
    kern_full = _edge_kernel(kernel_basis, Wk)
    partials = _sc_segment(x, kern_full, src, dst)
    return _ln_mlp(partials, x, conv_bias, ln_g, ln_b,
                   W1.T, b1, W2.T, b2, layer_scale)


# speedup vs baseline: 3.5453x; 1.5872x over previous
import functools

import jax
import jax.numpy as jnp
from jax import lax
from jax.experimental import pallas as pl
from jax.experimental.pallas import tpu as pltpu
from jax.experimental.pallas import tpu_sc as plsc

N = 10000
E = 320000
C = 128
K = 16
WF = 4 * C

NC = 2
NS = 16
NW = NC * NS
EPW = E // NW
CH = 80
NCHUNK = EPW // CH
N2 = 10240
RPT = N2 // NS
ZB = 16

_LANES = C // 16


def _edge_kernel(kernel_basis, Wk):
    e8 = E // 8
    basis2 = kernel_basis.reshape(e8, 8 * K)
    wbig = jnp.kron(jnp.eye(8, dtype=jnp.float32), Wk.T)

    def body(a_ref, w_ref, o_ref):
        o_ref[...] = jnp.dot(a_ref[...], w_ref[...],
                             preferred_element_type=jnp.float32)

    mb = 2000
    out = pl.pallas_call(
        body,
        grid=(e8 // mb,),
        in_specs=[
            pl.BlockSpec((mb, 8 * K), lambda i: (i, 0)),
            pl.BlockSpec((8 * K, 8 * C), lambda i: (0, 0)),
        ],
        out_specs=pl.BlockSpec((mb, 8 * C), lambda i: (i, 0)),
        out_shape=jax.ShapeDtypeStruct((e8, 8 * C), jnp.float32),
    )(basis2, wbig)
    return out.reshape(E, C)


def _sc_segment(x, kern_full, src, dst):
    mesh = plsc.VectorSubcoreMesh(core_axis_name="c", subcore_axis_name="s")

    @functools.partial(
        pl.kernel,
        out_type=jax.ShapeDtypeStruct((NC, N2, C), jnp.float32),
        mesh=mesh,
        scratch_types=[
            pltpu.VMEM((4, CH), jnp.int32),
            pltpu.VMEM((4, CH), jnp.int32),
            pltpu.VMEM((2, CH, C), jnp.float32),
            pltpu.VMEM((2, CH, C), jnp.float32),
            pltpu.VMEM((ZB, C), jnp.float32),
            pltpu.VMEM_SHARED((N2, C), jnp.float32),
            pltpu.SemaphoreType.DMA((4,)),
            pltpu.SemaphoreType.DMA((2,)),
            pltpu.SemaphoreType.DMA((2,)),
        ],
    )
    def sc(x_hbm, kern_hbm, src_hbm, dst_hbm, out_hbm,
           src_v, dst_v, xs_v, kern_v, zero_v, acc_sh,
           sem_i, sem_k, sem_g):
        cid = lax.axis_index("c")
        sid = lax.axis_index("s")
        wid = sid * NC + cid
        e0 = wid * EPW

        z16 = jnp.zeros((16,), jnp.float32)

        def zrow(i, carry):
            for c in range(_LANES):
                zero_v[i, pl.ds(c * 16, 16)] = z16
            return carry

        lax.fori_loop(0, ZB, zrow, 0)
        for r in range(RPT // ZB):
            pltpu.sync_copy(zero_v, acc_sh.at[pl.ds(sid * RPT + r * ZB, ZB)])
        plsc.subcore_barrier()

        def issue_idx(t, ib):
            sl = pl.ds(e0 + t * CH, CH)
            pltpu.async_copy(src_hbm.at[sl], src_v.at[ib], sem_i.at[ib])
            pltpu.async_copy(dst_hbm.at[sl], dst_v.at[ib], sem_i.at[ib])

        def wait_idx(t, ib):
            sl = pl.ds(e0 + t * CH, CH)
            pltpu.make_async_copy(src_hbm.at[sl], src_v.at[ib],
                                  sem_i.at[ib]).wait()
            pltpu.make_async_copy(dst_hbm.at[sl], dst_v.at[ib],
                                  sem_i.at[ib]).wait()

        def issue_kern(t, b):
            sl = pl.ds(e0 + t * CH, CH)
            pltpu.async_copy(kern_hbm.at[sl], kern_v.at[b], sem_k.at[b])

        def wait_kern(t, b):
            sl = pl.ds(e0 + t * CH, CH)
            pltpu.make_async_copy(kern_hbm.at[sl], kern_v.at[b],
                                  sem_k.at[b]).wait()

        def issue_gather(b, ib):
            pltpu.async_copy(x_hbm.at[src_v.at[ib]], xs_v.at[b], sem_g.at[b])

        def wait_gather(b, ib):
            pltpu.make_async_copy(x_hbm.at[src_v.at[ib]], xs_v.at[b],
                                  sem_g.at[b]).wait()

        for t in (0, 1):
            issue_idx(t, t)
            issue_kern(t, t)
        wait_idx(0, 0)
        issue_gather(0, 0)

        def chunk(j, carry):
            b = lax.rem(j, 2)
            ib = lax.rem(j, 4)
            bn = lax.rem(j + 1, 2)
            ibn = lax.rem(j + 1, 4)

            @pl.when(j + 2 < NCHUNK)
            def _():
                issue_idx(j + 2, lax.rem(j + 2, 4))

            @pl.when(j + 1 < NCHUNK)
            def _():
                wait_idx(j + 1, ibn)
                issue_gather(bn, ibn)

            wait_kern(j, b)
            wait_gather(b, ib)

            @plsc.parallel_loop(0, CH, 1, unroll=4)
            def erow(e):
                for c in range(_LANES):
                    sl = pl.ds(c * 16, 16)
                    xs_v[b, e, sl] = xs_v[b, e, sl] * kern_v[b, e, sl]

            @pl.when(j + 2 < NCHUNK)
            def _():
                issue_kern(j + 2, b)

            pltpu.sync_copy(xs_v.at[b], acc_sh.at[dst_v.at[ib]], add=True)
            return carry

        lax.fori_loop(0, NCHUNK, chunk, 0)
        plsc.subcore_barrier()

        for r in range(RPT // ZB):
            off = sid * RPT + r * ZB
            pltpu.sync_copy(acc_sh.at[pl.ds(off, ZB)],
                            out_hbm.at[cid, pl.ds(off, ZB)])

    return sc(x, kern_full, src, dst)


def _ln_mlp(partials, x, conv_bias, ln_g, ln_b, W1T, b1, W2T, b2, layer_scale):
    nb = 1000

    def body(p_ref, x_ref, cb_ref, g_ref, b_ref, w1_ref, b1_ref,
             w2_ref, b2_ref, ls_ref, o_ref):
        x1 = p_ref[0] + p_ref[1] + cb_ref[...]
        mu = jnp.mean(x1, axis=-1, keepdims=True)
        xc = x1 - mu
        var = jnp.mean(xc * xc, axis=-1, keepdims=True)
        h = xc * lax.rsqrt(var + 1e-5) * g_ref[...] + b_ref[...]
        a = jnp.dot(h, w1_ref[...], preferred_element_type=jnp.float32)
        a = a + b1_ref[...]
        a = 0.5 * a * (1.0 + lax.erf(a * 0.7071067811865476))
        o = jnp.dot(a, w2_ref[...], preferred_element_type=jnp.float32)
        o = o + b2_ref[...]
        o_ref[...] = ls_ref[...] * o + x_ref[...]

    return pl.pallas_call(
        body,
        grid=(N // nb,),
        in_specs=[
            pl.BlockSpec((NC, nb, C), lambda i: (0, i, 0)),
            pl.BlockSpec((nb, C), lambda i: (i, 0)),
            pl.BlockSpec((1, C), lambda i: (0, 0)),
            pl.BlockSpec((1, C), lambda i: (0, 0)),
            pl.BlockSpec((1, C), lambda i: (0, 0)),
            pl.BlockSpec((C, WF), lambda i: (0, 0)),
            pl.BlockSpec((1, WF), lambda i: (0, 0)),
            pl.BlockSpec((WF, C), lambda i: (0, 0)),
            pl.BlockSpec((1, C), lambda i: (0, 0)),
            pl.BlockSpec((1, C), lambda i: (0, 0)),
        ],
        out_specs=pl.BlockSpec((nb, C), lambda i: (i, 0)),
        out_shape=jax.ShapeDtypeStruct((N, C), jnp.float32),
    )(partials, x, conv_bias.reshape(1, C), ln_g.reshape(1, C),
      ln_b.reshape(1, C), W1T, b1.reshape(1, WF), W2T, b2.reshape(1, C),
      layer_scale.reshape(1, C))


def kernel(x, kernel_basis, fiber_kernel_basis, edge_index, Wk, conv_bias,
           ln_g, ln_b, W1, b1, W2, b2, layer_scale):
    src = edge_index[0]
    dst = edge_index[1]
    kern_full = _edge_kernel(kernel_basis, Wk)
    partials = _sc_segment(x, kern_full, src, dst)
    return _ln_mlp(partials, x, conv_bias, ln_g, ln_b,
                   W1.T, b1, W2.T, b2, layer_scale)


# speedup vs baseline: 4.7757x; 1.3471x over previous
import functools

import jax
import jax.numpy as jnp
from jax import lax
from jax.experimental import pallas as pl
from jax.experimental.pallas import tpu as pltpu
from jax.experimental.pallas import tpu_sc as plsc

N = 10000
E = 320000
C = 128
K = 16
WF = 4 * C

NC = 2
NS = 16
NW = NC * NS
EPW = E // NW
CH = 80
NCHUNK = EPW // CH
N2 = 10240
RPT = N2 // NS
ZB = 16

_LANES = C // 16


def _edge_kernel(kernel_basis, Wk):
    def body(a_ref, w_ref, o_ref):
        o_ref[...] = jnp.dot(a_ref[...], w_ref[...],
                             preferred_element_type=jnp.float32)

    eb = 8000
    return pl.pallas_call(
        body,
        grid=(E // eb,),
        in_specs=[
            pl.BlockSpec((eb, K), lambda i: (i, 0)),
            pl.BlockSpec((K, C), lambda i: (0, 0)),
        ],
        out_specs=pl.BlockSpec((eb, C), lambda i: (i, 0)),
        out_shape=jax.ShapeDtypeStruct((E, C), jnp.float32),
    )(kernel_basis, Wk.T)


def _sc_segment(x, kern_full, src, dst):
    mesh = plsc.VectorSubcoreMesh(core_axis_name="c", subcore_axis_name="s")

    @functools.partial(
        pl.kernel,
        out_type=jax.ShapeDtypeStruct((NC, N2, C), jnp.float32),
        mesh=mesh,
        scratch_types=[
            pltpu.VMEM((4, CH), jnp.int32),
            pltpu.VMEM((4, CH), jnp.int32),
            pltpu.VMEM((2, CH, C), jnp.float32),
            pltpu.VMEM((2, CH, C), jnp.float32),
            pltpu.VMEM((ZB, C), jnp.float32),
            pltpu.VMEM_SHARED((N2, C), jnp.float32),
            pltpu.SemaphoreType.DMA((4,)),
            pltpu.SemaphoreType.DMA((2,)),
            pltpu.SemaphoreType.DMA((2,)),
        ],
    )
    def sc(x_hbm, kern_hbm, src_hbm, dst_hbm, out_hbm,
           src_v, dst_v, xs_v, kern_v, zero_v, acc_sh,
           sem_i, sem_k, sem_g):
        cid = lax.axis_index("c")
        sid = lax.axis_index("s")
        wid = sid * NC + cid
        e0 = wid * EPW

        z16 = jnp.zeros((16,), jnp.float32)

        def zrow(i, carry):
            for c in range(_LANES):
                zero_v[i, pl.ds(c * 16, 16)] = z16
            return carry

        lax.fori_loop(0, ZB, zrow, 0)
        for r in range(RPT // ZB):
            pltpu.sync_copy(zero_v, acc_sh.at[pl.ds(sid * RPT + r * ZB, ZB)])
        plsc.subcore_barrier()

        def issue_idx(t, ib):
            sl = pl.ds(e0 + t * CH, CH)
            pltpu.async_copy(src_hbm.at[sl], src_v.at[ib], sem_i.at[ib])
            pltpu.async_copy(dst_hbm.at[sl], dst_v.at[ib], sem_i.at[ib])

        def wait_idx(t, ib):
            sl = pl.ds(e0 + t * CH, CH)
            pltpu.make_async_copy(src_hbm.at[sl], src_v.at[ib],
                                  sem_i.at[ib]).wait()
            pltpu.make_async_copy(dst_hbm.at[sl], dst_v.at[ib],
                                  sem_i.at[ib]).wait()

        def issue_kern(t, b):
            sl = pl.ds(e0 + t * CH, CH)
            pltpu.async_copy(kern_hbm.at[sl], kern_v.at[b], sem_k.at[b])

        def wait_kern(t, b):
            sl = pl.ds(e0 + t * CH, CH)
            pltpu.make_async_copy(kern_hbm.at[sl], kern_v.at[b],
                                  sem_k.at[b]).wait()

        def issue_gather(b, ib):
            pltpu.async_copy(x_hbm.at[src_v.at[ib]], xs_v.at[b], sem_g.at[b])

        def wait_gather(b, ib):
            pltpu.make_async_copy(x_hbm.at[src_v.at[ib]], xs_v.at[b],
                                  sem_g.at[b]).wait()

        for t in (0, 1):
            issue_idx(t, t)
            issue_kern(t, t)
        wait_idx(0, 0)
        issue_gather(0, 0)

        def chunk(j, carry):
            b = lax.rem(j, 2)
            ib = lax.rem(j, 4)
            bn = lax.rem(j + 1, 2)
            ibn = lax.rem(j + 1, 4)

            @pl.when(j + 2 < NCHUNK)
            def _():
                issue_idx(j + 2, lax.rem(j + 2, 4))

            @pl.when(j + 1 < NCHUNK)
            def _():
                wait_idx(j + 1, ibn)
                issue_gather(bn, ibn)

            wait_kern(j, b)
            wait_gather(b, ib)

            @plsc.parallel_loop(0, CH, 1, unroll=4)
            def erow(e):
                for c in range(_LANES):
                    sl = pl.ds(c * 16, 16)
                    xs_v[b, e, sl] = xs_v[b, e, sl] * kern_v[b, e, sl]

            @pl.when(j + 2 < NCHUNK)
            def _():
                issue_kern(j + 2, b)

            pltpu.sync_copy(xs_v.at[b], acc_sh.at[dst_v.at[ib]], add=True)
            return carry

        lax.fori_loop(0, NCHUNK, chunk, 0)
        plsc.subcore_barrier()

        for r in range(RPT // ZB):
            off = sid * RPT + r * ZB
            pltpu.sync_copy(acc_sh.at[pl.ds(off, ZB)],
                            out_hbm.at[cid, pl.ds(off, ZB)])

    return sc(x, kern_full, src, dst)


def _ln_mlp(partials, x, conv_bias, ln_g, ln_b, W1T, b1, W2T, b2, layer_scale):
    nb = 1000

    def body(p_ref, x_ref, cb_ref, g_ref, b_ref, w1_ref, b1_ref,
             w2_ref, b2_ref, ls_ref, o_ref):
        x1 = p_ref[0] + p_ref[1] + cb_ref[...]
        mu = jnp.mean(x1, axis=-1, keepdims=True)
        xc = x1 - mu
        var = jnp.mean(xc * xc, axis=-1, keepdims=True)
        h = xc * lax.rsqrt(var + 1e-5) * g_ref[...] + b_ref[...]
        a = jnp.dot(h, w1_ref[...], preferred_element_type=jnp.float32)
        a = a + b1_ref[...]
        a = 0.5 * a * (1.0 + lax.erf(a * 0.7071067811865476))
        o = jnp.dot(a, w2_ref[...], preferred_element_type=jnp.float32)
        o = o + b2_ref[...]
        o_ref[...] = ls_ref[...] * o + x_ref[...]

    return pl.pallas_call(
        body,
        grid=(N // nb,),
        in_specs=[
            pl.BlockSpec((NC, nb, C), lambda i: (0, i, 0)),
            pl.BlockSpec((nb, C), lambda i: (i, 0)),
            pl.BlockSpec((1, C), lambda i: (0, 0)),
            pl.BlockSpec((1, C), lambda i: (0, 0)),
            pl.BlockSpec((1, C), lambda i: (0, 0)),
            pl.BlockSpec((C, WF), lambda i: (0, 0)),
            pl.BlockSpec((1, WF), lambda i: (0, 0)),
            pl.BlockSpec((WF, C), lambda i: (0, 0)),
            pl.BlockSpec((1, C), lambda i: (0, 0)),
            pl.BlockSpec((1, C), lambda i: (0, 0)),
        ],
        out_specs=pl.BlockSpec((nb, C), lambda i: (i, 0)),
        out_shape=jax.ShapeDtypeStruct((N, C), jnp.float32),
    )(partials, x, conv_bias.reshape(1, C), ln_g.reshape(1, C),
      ln_b.reshape(1, C), W1T, b1.reshape(1, WF), W2T, b2.reshape(1, C),
      layer_scale.reshape(1, C))


def kernel(x, kernel_basis, fiber_kernel_basis, edge_index, Wk, conv_bias,
           ln_g, ln_b, W1, b1, W2, b2, layer_scale):
    src = edge_index[0]
    dst = edge_index[1]
    kern_full = _edge_kernel(kernel_basis, Wk)
    partials = _sc_segment(x, kern_full, src, dst)
    return _ln_mlp(partials, x, conv_bias, ln_g, ln_b,
                   W1.T, b1, W2.T, b2, layer_scale)
